# baseline (speedup 1.0000x reference)
import functools

import jax
import jax.numpy as jnp
from jax import lax
from jax.experimental import pallas as pl
from jax.experimental.pallas import tpu as pltpu
from jax.experimental.pallas import tpu_sc as plsc

NS = 16
LANES = 16
CHUNK = 48
SUP = 768
NCH = SUP // CHUNK
BLK = 2048


def _tc1_body(x_ref, w_ref, as_ref, ad_ref, xp_ref, asrc_ref, adst_ref):
    xb = jnp.dot(x_ref[...], w_ref[...], preferred_element_type=jnp.float32)
    xp_ref[...] = xb
    ss, dd = [], []
    for h in range(2):
        blk = xb[:, h * 128:(h + 1) * 128]
        ss.append(jnp.sum(blk * as_ref[h, :][None, :], axis=1)[None, :])
        dd.append(jnp.sum(blk * ad_ref[h, :][None, :], axis=1)[None, :])
    asrc_ref[...] = jnp.concatenate(ss, axis=0)
    adst_ref[...] = jnp.concatenate(dd, axis=0)


def _tc1(x, W, att_s, att_d):
    n = x.shape[0]
    f = x.shape[1]
    return pl.pallas_call(
        _tc1_body,
        grid=(pl.cdiv(n, BLK),),
        in_specs=[
            pl.BlockSpec((BLK, f), lambda i: (i, 0)),
            pl.BlockSpec((f, 256), lambda i: (0, 0)),
            pl.BlockSpec((2, 128), lambda i: (0, 0)),
            pl.BlockSpec((2, 128), lambda i: (0, 0)),
        ],
        out_specs=[
            pl.BlockSpec((BLK, 256), lambda i: (i, 0)),
            pl.BlockSpec((2, BLK), lambda i: (0, i)),
            pl.BlockSpec((2, BLK), lambda i: (0, i)),
        ],
        out_shape=[
            jax.ShapeDtypeStruct((n, 256), jnp.float32),
            jax.ShapeDtypeStruct((2, n), jnp.float32),
            jax.ShapeDtypeStruct((2, n), jnp.float32),
        ],
    )(x, W, att_s, att_d)


def _sc_edge(src_p, dst_p, asrc, adst, xp2, zrows, n, e_real, ept):
    nsup = ept // SUP
    rpt = ((n // NS + 7) // 8) * 8
    npad = NS * rpt
    mesh = plsc.VectorSubcoreMesh(core_axis_name="c", subcore_axis_name="s")

    @functools.partial(
        pl.kernel,
        mesh=mesh,
        compiler_params=pltpu.CompilerParams(needs_layout_passes=False),
        out_type=(
            jax.ShapeDtypeStruct((2, npad, 128), jnp.float32),
            jax.ShapeDtypeStruct((2, NS, n), jnp.float32),
        ),
        scratch_types=[
            pltpu.VMEM((n,), jnp.float32),
            pltpu.VMEM((n,), jnp.float32),
            pltpu.VMEM((n,), jnp.float32),
            pltpu.VMEM((NCH, CHUNK), jnp.int32),
            pltpu.VMEM((NCH, CHUNK), jnp.int32),
            pltpu.VMEM((CHUNK,), jnp.int32),
            pltpu.VMEM((CHUNK,), jnp.int32),
            pltpu.VMEM((CHUNK,), jnp.int32),
            pltpu.VMEM((CHUNK,), jnp.int32),
            pltpu.VMEM((CHUNK, 128), jnp.float32),
            pltpu.VMEM((CHUNK, 128), jnp.float32),
            pltpu.VMEM((CHUNK,), jnp.float32),
            pltpu.VMEM((LANES,), jnp.int32),
            pltpu.VMEM((LANES,), jnp.float32),
            pltpu.VMEM_SHARED((npad, 128), jnp.float32),
            pltpu.SemaphoreType.DMA,
            pltpu.SemaphoreType.DMA,
        ],
    )
    def body(src_h, dst_h, asrc_h, adst_h, xp2_h, zr_h, msg_h, denp_h,
             asrc_t, adst_t, den_t, src_t, dst_t, gidx0, gidx1,
             didx0, didx1, rows0, rows1, w_t, kbuf, wbuf, acc_s,
             sem_g, sem_s):
        cid = lax.axis_index("c")
        sid = lax.axis_index("s")
        e0 = sid * ept
        pltpu.sync_copy(asrc_h.at[cid], asrc_t)
        pltpu.sync_copy(adst_h.at[cid], adst_t)
        pltpu.sync_copy(zr_h, acc_s.at[pl.ds(sid * rpt, rpt)])

        zv = jnp.zeros((LANES,), jnp.float32)

        def zb(i, _):
            den_t[pl.ds(i * LANES, LANES)] = zv
            return 0
        lax.fori_loop(0, n // LANES, zb, 0)
        plsc.subcore_barrier()

        cvec = jnp.full((LANES,), cid, dtype=jnp.int32)
        iota = lax.iota(jnp.int32, LANES)
        shifts = [1, 2, 4, 8]
        shift_idx = [jnp.maximum(iota - s, 0) for s in shifts]
        next_idx = jnp.minimum(iota + 1, LANES - 1)
        gbufs = [gidx0, gidx1]
        dbufs = [didx0, didx1]
        rbufs = [rows0, rows1]

        def build_idx(j):
            gb, db = gbufs[j % 2], dbufs[j % 2]
            for k in range(CHUNK // LANES):
                sv = src_t[j, pl.ds(k * LANES, LANES)]
                dv = dst_t[j, pl.ds(k * LANES, LANES)]
                gb[pl.ds(k * LANES, LANES)] = sv * 2 + cvec
                db[pl.ds(k * LANES, LANES)] = dv

        def fire_gather(j):
            pltpu.async_copy(xp2_h.at[gbufs[j % 2]], rbufs[j % 2], sem_g)

        def wait_gather(j):
            pltpu.make_async_copy(
                xp2_h.at[gbufs[j % 2]], rbufs[j % 2], sem_g).wait()

        def fire_scatter(j):
            pltpu.async_copy(rbufs[j % 2], acc_s.at[dbufs[j % 2]],
                             sem_s, add=True)

        def wait_scatter(j):
            pltpu.make_async_copy(rbufs[j % 2], acc_s.at[dbufs[j % 2]],
                                  sem_s).wait()

        def compute_wden(s, j):
            for k in range(CHUNK // LANES):
                sv = src_t[j, pl.ds(k * LANES, LANES)]
                dv = dst_t[j, pl.ds(k * LANES, LANES)]
                al = plsc.load_gather(asrc_t, [sv]) + \
                    plsc.load_gather(adst_t, [dv])
                al = jnp.where(al > 0, al, al * jnp.float32(0.2))
                w = jnp.exp(al)
                egid = e0 + s * SUP + j * CHUNK + k * LANES + iota
                w = jnp.where(egid < e_real, w, jnp.float32(0.0))
                w_t[pl.ds(k * LANES, LANES)] = w
                ks_, vs_ = plsc.sort_key_val(dv, w)
                kbuf[...] = ks_
                for si, sh in enumerate(shifts):
                    wbuf[...] = vs_
                    kprev = plsc.load_gather(kbuf, [shift_idx[si]])
                    vprev = plsc.load_gather(wbuf, [shift_idx[si]])
                    ok = (iota >= sh) & (kprev == ks_)
                    vs_ = vs_ + jnp.where(ok, vprev, jnp.float32(0.0))
                knext = plsc.load_gather(kbuf, [next_idx])
                ends = (iota == LANES - 1) | (knext != ks_)
                plsc.addupdate_scatter(den_t, [ks_], vs_, mask=ends)

        def scale_rows(j):
            rb = rbufs[j % 2]

            @plsc.parallel_loop(0, CHUNK, step=1, unroll=4)
            def scale_row(r):
                wv = plsc.load_gather(
                    w_t, [jnp.full((LANES,), r, dtype=jnp.int32)])
                for jj in range(128 // LANES):
                    sl = pl.ds(jj * LANES, LANES)
                    rb[r, sl] = rb[r, sl] * wv

        def sup_body(s, _):
            row0 = (sid * nsup + s) * NCH
            pltpu.sync_copy(src_h.at[pl.ds(row0, NCH)], src_t)
            pltpu.sync_copy(dst_h.at[pl.ds(row0, NCH)], dst_t)
            for j in range(NCH):
                if j == 0:
                    build_idx(0)
                    fire_gather(0)
                compute_wden(s, j)
                if j > 0:
                    wait_scatter(j - 1)
                if j < NCH - 1:
                    build_idx(j + 1)
                    fire_gather(j + 1)
                wait_gather(j)
                scale_rows(j)
                fire_scatter(j)
            wait_scatter(NCH - 1)
            return 0

        lax.fori_loop(0, nsup, sup_body, 0)
        plsc.subcore_barrier()
        pltpu.sync_copy(acc_s.at[pl.ds(sid * rpt, rpt)],
                        msg_h.at[cid, pl.ds(sid * rpt, rpt)])
        pltpu.sync_copy(den_t, denp_h.at[cid, sid])

    return body(src_p, dst_p, asrc, adst, xp2, zrows)


def _tc2_body(msg_ref, denp_ref, bc_ref, wa_ref, ba_ref, w1_ref, b1_ref,
              w2_ref, b2_ref, w3_ref, b3_ref, h3_ref):
    heads = []
    for h in range(2):
        den = jnp.sum(denp_ref[h], axis=0) + jnp.float32(1e-16)
        heads.append(msg_ref[h] / den[:, None])
    g = jnp.concatenate(heads, axis=1) + bc_ref[...]

    def lrelu(t):
        return jnp.where(t > 0, t, t * jnp.float32(0.01))

    g = lrelu(g)
    g = lrelu(jnp.dot(g, wa_ref[...], preferred_element_type=jnp.float32)
              + ba_ref[...])
    g = lrelu(jnp.dot(g, w1_ref[...], preferred_element_type=jnp.float32)
              + b1_ref[...])
    g = lrelu(jnp.dot(g, w2_ref[...], preferred_element_type=jnp.float32)
              + b2_ref[...])
    h3_ref[...] = jnp.dot(g, w3_ref[...], preferred_element_type=jnp.float32) \
        + b3_ref[...]


def _tc2(msg, denp, b_conv, Wa, ba, W1, b1, W2, b2, W3, b3):
    n = denp.shape[2]
    full = lambda shape: pl.BlockSpec(shape, lambda i: tuple(0 for _ in shape))
    return pl.pallas_call(
        _tc2_body,
        grid=(pl.cdiv(n, BLK),),
        in_specs=[
            pl.BlockSpec((2, BLK, 128), lambda i: (0, i, 0)),
            pl.BlockSpec((2, NS, BLK), lambda i: (0, 0, i)),
            full((1, 256)),
            full((256, 128)), full((1, 128)),
            full((128, 64)), full((1, 64)),
            full((64, 32)), full((1, 32)),
            full((32, 3)), full((1, 3)),
        ],
        out_specs=pl.BlockSpec((BLK, 3), lambda i: (i, 0)),
        out_shape=jax.ShapeDtypeStruct((n, 3), jnp.float32),
    )(msg, denp, b_conv.reshape(1, 256), Wa, ba.reshape(1, -1),
      W1, b1.reshape(1, -1), W2, b2.reshape(1, -1),
      W3, b3.reshape(1, -1))


def _tc3_body(hi_ref, hj_ref, out_ref):
    hi = hi_ref[...]
    hj = hj_ref[...]
    x2i = jnp.sum(hi * hi, axis=1)
    x2j = jnp.sum(hj * hj, axis=1)
    ip = lax.dot_general(hi, hj, (((1,), (1,)), ((), ())),
                         preferred_element_type=jnp.float32)
    d2 = x2i[:, None] + x2j[None, :] - 2.0 * ip
    d2 = jnp.maximum(d2, 0.0)
    pos = d2 > 0
    out_ref[...] = jnp.where(pos, jnp.sqrt(jnp.where(pos, d2, 1.0)), 0.0)


def _tc3(h3):
    n = h3.shape[0]
    br, bc = 256, 2048
    return pl.pallas_call(
        _tc3_body,
        grid=(pl.cdiv(n, br), pl.cdiv(n, bc)),
        in_specs=[
            pl.BlockSpec((br, 3), lambda i, j: (i, 0)),
            pl.BlockSpec((bc, 3), lambda i, j: (j, 0)),
        ],
        out_specs=pl.BlockSpec((br, bc), lambda i, j: (i, j)),
        out_shape=jax.ShapeDtypeStruct((n, n), jnp.float32),
    )(h3, h3)


def kernel(x, edge_index, W, att_src, att_dst, b_conv,
           Wa, ba, W1, b1, W2, b2, W3, b3):
    n = x.shape[0]
    e = edge_index.shape[1]

    xp, asrc, adst = _tc1(x, W, att_src.reshape(2, 128),
                          att_dst.reshape(2, 128))
    xp2 = xp.reshape(2 * n, 128)

    ept = ((e + NS * SUP - 1) // (NS * SUP)) * SUP
    e_pad = NS * ept
    src = edge_index[0].astype(jnp.int32)
    dst = edge_index[1].astype(jnp.int32)
    pad = jnp.zeros((e_pad - e,), dtype=jnp.int32)
    src_p = jnp.concatenate([src, pad]).reshape(-1, CHUNK)
    dst_p = jnp.concatenate([dst, pad]).reshape(-1, CHUNK)
    rpt = ((n // NS + 7) // 8) * 8
    zrows = jnp.zeros((rpt, 128), dtype=jnp.float32)

    msg, denp = _sc_edge(src_p, dst_p, asrc, adst, xp2, zrows, n, e, ept)

    h3 = _tc2(msg, denp, b_conv, Wa, ba, W1, b1, W2, b2, W3, b3)
    return _tc3(h3)


# speedup vs baseline: 1.5809x; 1.5809x over previous
import functools

import jax
import jax.numpy as jnp
from jax import lax
from jax.experimental import pallas as pl
from jax.experimental.pallas import tpu as pltpu
from jax.experimental.pallas import tpu_sc as plsc

NS = 16
LANES = 16
CHUNK = 128
NCH = 8
SUP = NCH * CHUNK
BLK = 2048


def _tc1_body(x_ref, w_ref, as_ref, ad_ref, xp_ref, asrc_ref, adst_ref):
    xb = jnp.dot(x_ref[...], w_ref[...], preferred_element_type=jnp.float32)
    xp_ref[...] = xb
    ss, dd = [], []
    for h in range(2):
        blk = xb[:, h * 128:(h + 1) * 128]
        ss.append(jnp.sum(blk * as_ref[h, :][None, :], axis=1)[None, :])
        dd.append(jnp.sum(blk * ad_ref[h, :][None, :], axis=1)[None, :])
    asrc_ref[...] = jnp.concatenate(ss, axis=0)
    adst_ref[...] = jnp.concatenate(dd, axis=0)


def _tc1(x, W, att_s, att_d):
    n = x.shape[0]
    f = x.shape[1]
    return pl.pallas_call(
        _tc1_body,
        grid=(pl.cdiv(n, BLK),),
        in_specs=[
            pl.BlockSpec((BLK, f), lambda i: (i, 0)),
            pl.BlockSpec((f, 256), lambda i: (0, 0)),
            pl.BlockSpec((2, 128), lambda i: (0, 0)),
            pl.BlockSpec((2, 128), lambda i: (0, 0)),
        ],
        out_specs=[
            pl.BlockSpec((BLK, 256), lambda i: (i, 0)),
            pl.BlockSpec((2, BLK), lambda i: (0, i)),
            pl.BlockSpec((2, BLK), lambda i: (0, i)),
        ],
        out_shape=[
            jax.ShapeDtypeStruct((n, 256), jnp.float32),
            jax.ShapeDtypeStruct((2, n), jnp.float32),
            jax.ShapeDtypeStruct((2, n), jnp.float32),
        ],
    )(x, W, att_s, att_d)


def _sc_weights(src_p, dst_p, asrc, adst, n, e_real, ept):
    nsup = ept // SUP
    mesh = plsc.VectorSubcoreMesh(core_axis_name="c", subcore_axis_name="s")

    @functools.partial(
        pl.kernel,
        mesh=mesh,
        compiler_params=pltpu.CompilerParams(needs_layout_passes=False),
        out_type=(
            jax.ShapeDtypeStruct((2, NS * ept // CHUNK, CHUNK), jnp.float32),
            jax.ShapeDtypeStruct((2, NS, n), jnp.float32),
        ),
        scratch_types=[
            pltpu.VMEM((n,), jnp.float32),
            pltpu.VMEM((n,), jnp.float32),
            pltpu.VMEM((n,), jnp.float32),
            pltpu.VMEM((NCH, CHUNK), jnp.int32),
            pltpu.VMEM((NCH, CHUNK), jnp.int32),
            pltpu.VMEM((NCH, CHUNK), jnp.float32),
            pltpu.VMEM((LANES,), jnp.int32),
            pltpu.VMEM((LANES,), jnp.float32),
        ],
    )
    def body(src_h, dst_h, asrc_h, adst_h, w_all_h, denp_h,
             asrc_t, adst_t, den_t, src_t, dst_t, w_sup, kbuf, wbuf):
        cid = lax.axis_index("c")
        sid = lax.axis_index("s")
        e0 = sid * ept
        pltpu.sync_copy(asrc_h.at[cid], asrc_t)
        pltpu.sync_copy(adst_h.at[cid], adst_t)

        zv = jnp.zeros((LANES,), jnp.float32)

        def zb(i, _):
            den_t[pl.ds(i * LANES, LANES)] = zv
            return 0
        lax.fori_loop(0, n // LANES, zb, 0)

        iota = lax.iota(jnp.int32, LANES)
        shifts = [1, 2, 4, 8]
        shift_idx = [jnp.maximum(iota - s, 0) for s in shifts]
        next_idx = jnp.minimum(iota + 1, LANES - 1)

        def sup_body(s, _):
            row0 = (sid * nsup + s) * NCH
            pltpu.sync_copy(src_h.at[pl.ds(row0, NCH)], src_t)
            pltpu.sync_copy(dst_h.at[pl.ds(row0, NCH)], dst_t)
            for j in range(NCH):
                for k in range(CHUNK // LANES):
                    sv = src_t[j, pl.ds(k * LANES, LANES)]
                    dv = dst_t[j, pl.ds(k * LANES, LANES)]
                    al = plsc.load_gather(asrc_t, [sv]) + \
                        plsc.load_gather(adst_t, [dv])
                    al = jnp.where(al > 0, al, al * jnp.float32(0.2))
                    w = jnp.exp(al)
                    egid = e0 + s * SUP + j * CHUNK + k * LANES + iota
                    w = jnp.where(egid < e_real, w, jnp.float32(0.0))
                    w_sup[j, pl.ds(k * LANES, LANES)] = w
                    ks_, vs_ = plsc.sort_key_val(dv, w)
                    kbuf[...] = ks_
                    for si, sh in enumerate(shifts):
                        wbuf[...] = vs_
                        kprev = plsc.load_gather(kbuf, [shift_idx[si]])
                        vprev = plsc.load_gather(wbuf, [shift_idx[si]])
                        ok = (iota >= sh) & (kprev == ks_)
                        vs_ = vs_ + jnp.where(ok, vprev, jnp.float32(0.0))
                    knext = plsc.load_gather(kbuf, [next_idx])
                    ends = (iota == LANES - 1) | (knext != ks_)
                    plsc.addupdate_scatter(den_t, [ks_], vs_, mask=ends)
            pltpu.sync_copy(w_sup, w_all_h.at[cid, pl.ds(row0, NCH)])
            return 0

        lax.fori_loop(0, nsup, sup_body, 0)
        pltpu.sync_copy(den_t, denp_h.at[cid, sid])

    return body(src_p, dst_p, asrc, adst)


def _sc_rows(src_p, dst_p, w_all, xp2, zrows, n, ept):
    nsup = ept // SUP
    rpt = ((n // NS + 7) // 8) * 8
    npad = NS * rpt
    mesh = plsc.VectorSubcoreMesh(core_axis_name="c", subcore_axis_name="s")

    @functools.partial(
        pl.kernel,
        mesh=mesh,
        compiler_params=pltpu.CompilerParams(needs_layout_passes=False),
        out_type=jax.ShapeDtypeStruct((2, npad, 128), jnp.float32),
        scratch_types=[
            pltpu.VMEM((NCH, CHUNK), jnp.int32),
            pltpu.VMEM((NCH, CHUNK), jnp.int32),
            pltpu.VMEM((NCH, CHUNK), jnp.float32),
            pltpu.VMEM((CHUNK,), jnp.int32),
            pltpu.VMEM((CHUNK,), jnp.int32),
            pltpu.VMEM((CHUNK,), jnp.int32),
            pltpu.VMEM((CHUNK,), jnp.int32),
            pltpu.VMEM((CHUNK, 128), jnp.float32),
            pltpu.VMEM((CHUNK, 128), jnp.float32),
            pltpu.VMEM_SHARED((npad, 128), jnp.float32),
            pltpu.SemaphoreType.DMA,
            pltpu.SemaphoreType.DMA,
        ],
    )
    def body(src_h, dst_h, w_all_h, xp2_h, zr_h, msg_h,
             src_t, dst_t, w_sup, gidx0, gidx1, didx0, didx1,
             rows0, rows1, acc_s, sem_g, sem_s):
        cid = lax.axis_index("c")
        sid = lax.axis_index("s")
        pltpu.sync_copy(zr_h, acc_s.at[pl.ds(sid * rpt, rpt)])
        plsc.subcore_barrier()

        cvec = jnp.full((LANES,), cid, dtype=jnp.int32)
        gbufs = [gidx0, gidx1]
        dbufs = [didx0, didx1]
        rbufs = [rows0, rows1]

        def build_idx(j):
            gb, db = gbufs[j % 2], dbufs[j % 2]
            for k in range(CHUNK // LANES):
                sv = src_t[j, pl.ds(k * LANES, LANES)]
                dv = dst_t[j, pl.ds(k * LANES, LANES)]
                gb[pl.ds(k * LANES, LANES)] = sv * 2 + cvec
                db[pl.ds(k * LANES, LANES)] = dv

        def fire_gather(j):
            pltpu.async_copy(xp2_h.at[gbufs[j % 2]], rbufs[j % 2], sem_g)

        def wait_gather(j):
            pltpu.make_async_copy(
                xp2_h.at[gbufs[j % 2]], rbufs[j % 2], sem_g).wait()

        def fire_scatter(j):
            pltpu.async_copy(rbufs[j % 2], acc_s.at[dbufs[j % 2]],
                             sem_s, add=True)

        def wait_scatter(j):
            pltpu.make_async_copy(rbufs[j % 2], acc_s.at[dbufs[j % 2]],
                                  sem_s).wait()

        def scale_rows(j):
            rb = rbufs[j % 2]

            @plsc.parallel_loop(0, CHUNK, step=1, unroll=4)
            def scale_row(r):
                ri = jnp.full((LANES,), r, dtype=jnp.int32)
                cj = jnp.full((LANES,), j, dtype=jnp.int32)
                wv = plsc.load_gather(w_sup, [cj, ri])
                for jj in range(128 // LANES):
                    sl = pl.ds(jj * LANES, LANES)
                    rb[r, sl] = rb[r, sl] * wv

        def sup_body(s, _):
            row0 = (sid * nsup + s) * NCH
            pltpu.sync_copy(src_h.at[pl.ds(row0, NCH)], src_t)
            pltpu.sync_copy(dst_h.at[pl.ds(row0, NCH)], dst_t)
            pltpu.sync_copy(w_all_h.at[cid, pl.ds(row0, NCH)], w_sup)
            for j in range(NCH):
                if j == 0:
                    build_idx(0)
                    fire_gather(0)
                if j > 0:
                    wait_scatter(j - 1)
                if j < NCH - 1:
                    build_idx(j + 1)
                    fire_gather(j + 1)
                wait_gather(j)
                scale_rows(j)
                fire_scatter(j)
            wait_scatter(NCH - 1)
            return 0

        lax.fori_loop(0, nsup, sup_body, 0)
        plsc.subcore_barrier()
        pltpu.sync_copy(acc_s.at[pl.ds(sid * rpt, rpt)],
                        msg_h.at[cid, pl.ds(sid * rpt, rpt)])

    return body(src_p, dst_p, w_all, xp2, zrows)


def _tc2_body(msg_ref, denp_ref, bc_ref, wa_ref, ba_ref, w1_ref, b1_ref,
              w2_ref, b2_ref, w3_ref, b3_ref, h3_ref):
    heads = []
    for h in range(2):
        den = jnp.sum(denp_ref[h], axis=0) + jnp.float32(1e-16)
        heads.append(msg_ref[h] / den[:, None])
    g = jnp.concatenate(heads, axis=1) + bc_ref[...]

    def lrelu(t):
        return jnp.where(t > 0, t, t * jnp.float32(0.01))

    g = lrelu(g)
    g = lrelu(jnp.dot(g, wa_ref[...], preferred_element_type=jnp.float32)
              + ba_ref[...])
    g = lrelu(jnp.dot(g, w1_ref[...], preferred_element_type=jnp.float32)
              + b1_ref[...])
    g = lrelu(jnp.dot(g, w2_ref[...], preferred_element_type=jnp.float32)
              + b2_ref[...])
    h3_ref[...] = jnp.dot(g, w3_ref[...], preferred_element_type=jnp.float32) \
        + b3_ref[...]


def _tc2(msg, denp, b_conv, Wa, ba, W1, b1, W2, b2, W3, b3):
    n = denp.shape[2]
    full = lambda shape: pl.BlockSpec(shape, lambda i: tuple(0 for _ in shape))
    return pl.pallas_call(
        _tc2_body,
        grid=(pl.cdiv(n, BLK),),
        in_specs=[
            pl.BlockSpec((2, BLK, 128), lambda i: (0, i, 0)),
            pl.BlockSpec((2, NS, BLK), lambda i: (0, 0, i)),
            full((1, 256)),
            full((256, 128)), full((1, 128)),
            full((128, 64)), full((1, 64)),
            full((64, 32)), full((1, 32)),
            full((32, 3)), full((1, 3)),
        ],
        out_specs=pl.BlockSpec((BLK, 3), lambda i: (i, 0)),
        out_shape=jax.ShapeDtypeStruct((n, 3), jnp.float32),
    )(msg, denp, b_conv.reshape(1, 256), Wa, ba.reshape(1, -1),
      W1, b1.reshape(1, -1), W2, b2.reshape(1, -1),
      W3, b3.reshape(1, -1))


def _tc3_body(hi_ref, hj_ref, out_ref):
    hi = hi_ref[...]
    hj = hj_ref[...]
    x2i = jnp.sum(hi * hi, axis=1)
    x2j = jnp.sum(hj * hj, axis=1)
    ip = lax.dot_general(hi, hj, (((1,), (1,)), ((), ())),
                         preferred_element_type=jnp.float32)
    d2 = x2i[:, None] + x2j[None, :] - 2.0 * ip
    d2 = jnp.maximum(d2, 0.0)
    pos = d2 > 0
    out_ref[...] = jnp.where(pos, jnp.sqrt(jnp.where(pos, d2, 1.0)), 0.0)


def _tc3(h3):
    n = h3.shape[0]
    br, bc = 256, 2048
    return pl.pallas_call(
        _tc3_body,
        grid=(pl.cdiv(n, br), pl.cdiv(n, bc)),
        in_specs=[
            pl.BlockSpec((br, 3), lambda i, j: (i, 0)),
            pl.BlockSpec((bc, 3), lambda i, j: (j, 0)),
        ],
        out_specs=pl.BlockSpec((br, bc), lambda i, j: (i, j)),
        out_shape=jax.ShapeDtypeStruct((n, n), jnp.float32),
    )(h3, h3)


def kernel(x, edge_index, W, att_src, att_dst, b_conv,
           Wa, ba, W1, b1, W2, b2, W3, b3):
    n = x.shape[0]
    e = edge_index.shape[1]

    xp, asrc, adst = _tc1(x, W, att_src.reshape(2, 128),
                          att_dst.reshape(2, 128))
    xp2 = xp.reshape(2 * n, 128)

    ept = ((e + NS * SUP - 1) // (NS * SUP)) * SUP
    e_pad = NS * ept
    src = edge_index[0].astype(jnp.int32)
    dst = edge_index[1].astype(jnp.int32)
    pad = jnp.zeros((e_pad - e,), dtype=jnp.int32)
    src_p = jnp.concatenate([src, pad]).reshape(-1, CHUNK)
    dst_p = jnp.concatenate([dst, pad]).reshape(-1, CHUNK)
    rpt = ((n // NS + 7) // 8) * 8
    zrows = jnp.zeros((rpt, 128), dtype=jnp.float32)

    w_all, denp = _sc_weights(src_p, dst_p, asrc, adst, n, e, ept)
    msg = _sc_rows(src_p, dst_p, w_all, xp2, zrows, n, ept)

    h3 = _tc2(msg, denp, b_conv, Wa, ba, W1, b1, W2, b2, W3, b3)
    return _tc3(h3)


# speedup vs baseline: 1.5921x; 1.0071x over previous
import functools

import jax
import jax.numpy as jnp
from jax import lax
from jax.experimental import pallas as pl
from jax.experimental.pallas import tpu as pltpu
from jax.experimental.pallas import tpu_sc as plsc

NS = 16
LANES = 16
CHUNK = 128
NCH = 16
SUP = NCH * CHUNK
BLK = 2048


def _tc1_body(x_ref, w_ref, as_ref, ad_ref, xp_ref, asrc_ref, adst_ref):
    xb = jnp.dot(x_ref[...], w_ref[...], preferred_element_type=jnp.float32)
    xp_ref[...] = xb
    ss, dd = [], []
    for h in range(2):
        blk = xb[:, h * 128:(h + 1) * 128]
        ss.append(jnp.sum(blk * as_ref[h, :][None, :], axis=1)[None, :])
        dd.append(jnp.sum(blk * ad_ref[h, :][None, :], axis=1)[None, :])
    asrc_ref[...] = jnp.concatenate(ss, axis=0)
    adst_ref[...] = jnp.concatenate(dd, axis=0)


def _tc1(x, W, att_s, att_d):
    n = x.shape[0]
    f = x.shape[1]
    return pl.pallas_call(
        _tc1_body,
        grid=(pl.cdiv(n, BLK),),
        in_specs=[
            pl.BlockSpec((BLK, f), lambda i: (i, 0)),
            pl.BlockSpec((f, 256), lambda i: (0, 0)),
            pl.BlockSpec((2, 128), lambda i: (0, 0)),
            pl.BlockSpec((2, 128), lambda i: (0, 0)),
        ],
        out_specs=[
            pl.BlockSpec((BLK, 256), lambda i: (i, 0)),
            pl.BlockSpec((2, BLK), lambda i: (0, i)),
            pl.BlockSpec((2, BLK), lambda i: (0, i)),
        ],
        out_shape=[
            jax.ShapeDtypeStruct((n, 256), jnp.float32),
            jax.ShapeDtypeStruct((2, n), jnp.float32),
            jax.ShapeDtypeStruct((2, n), jnp.float32),
        ],
    )(x, W, att_s, att_d)


def _sc_weights(src_p, dst_p, asrc, adst, n, e_real, ept):
    nsup = ept // SUP
    mesh = plsc.VectorSubcoreMesh(core_axis_name="c", subcore_axis_name="s")

    @functools.partial(
        pl.kernel,
        mesh=mesh,
        compiler_params=pltpu.CompilerParams(needs_layout_passes=False),
        out_type=(
            jax.ShapeDtypeStruct((2, NS * ept // CHUNK, CHUNK), jnp.float32),
            jax.ShapeDtypeStruct((2, NS, n), jnp.float32),
        ),
        scratch_types=[
            pltpu.VMEM((n,), jnp.float32),
            pltpu.VMEM((n,), jnp.float32),
            pltpu.VMEM((n,), jnp.float32),
            pltpu.VMEM((NCH, CHUNK), jnp.int32),
            pltpu.VMEM((NCH, CHUNK), jnp.int32),
            pltpu.VMEM((NCH, CHUNK), jnp.float32),
            pltpu.VMEM((LANES,), jnp.int32),
            pltpu.VMEM((LANES,), jnp.float32),
        ],
    )
    def body(src_h, dst_h, asrc_h, adst_h, w_all_h, denp_h,
             asrc_t, adst_t, den_t, src_t, dst_t, w_sup, kbuf, wbuf):
        cid = lax.axis_index("c")
        sid = lax.axis_index("s")
        e0 = sid * ept
        pltpu.sync_copy(asrc_h.at[cid], asrc_t)
        pltpu.sync_copy(adst_h.at[cid], adst_t)

        zv = jnp.zeros((LANES,), jnp.float32)

        def zb(i, _):
            den_t[pl.ds(i * LANES, LANES)] = zv
            return 0
        lax.fori_loop(0, n // LANES, zb, 0)

        iota = lax.iota(jnp.int32, LANES)
        shifts = [1, 2, 4, 8]
        shift_idx = [jnp.maximum(iota - s, 0) for s in shifts]
        next_idx = jnp.minimum(iota + 1, LANES - 1)

        def sup_body(s, _):
            row0 = (sid * nsup + s) * NCH
            pltpu.sync_copy(src_h.at[pl.ds(row0, NCH)], src_t)
            pltpu.sync_copy(dst_h.at[pl.ds(row0, NCH)], dst_t)
            for j in range(NCH):
                for k in range(CHUNK // LANES):
                    sv = src_t[j, pl.ds(k * LANES, LANES)]
                    dv = dst_t[j, pl.ds(k * LANES, LANES)]
                    al = plsc.load_gather(asrc_t, [sv]) + \
                        plsc.load_gather(adst_t, [dv])
                    al = jnp.where(al > 0, al, al * jnp.float32(0.2))
                    w = jnp.exp(al)
                    egid = e0 + s * SUP + j * CHUNK + k * LANES + iota
                    w = jnp.where(egid < e_real, w, jnp.float32(0.0))
                    w_sup[j, pl.ds(k * LANES, LANES)] = w
                    ks_, vs_ = plsc.sort_key_val(dv, w)
                    kbuf[...] = ks_
                    for si, sh in enumerate(shifts):
                        wbuf[...] = vs_
                        kprev = plsc.load_gather(kbuf, [shift_idx[si]])
                        vprev = plsc.load_gather(wbuf, [shift_idx[si]])
                        ok = (iota >= sh) & (kprev == ks_)
                        vs_ = vs_ + jnp.where(ok, vprev, jnp.float32(0.0))
                    knext = plsc.load_gather(kbuf, [next_idx])
                    ends = (iota == LANES - 1) | (knext != ks_)
                    plsc.addupdate_scatter(den_t, [ks_], vs_, mask=ends)
            pltpu.sync_copy(w_sup, w_all_h.at[cid, pl.ds(row0, NCH)])
            return 0

        lax.fori_loop(0, nsup, sup_body, 0)
        pltpu.sync_copy(den_t, denp_h.at[cid, sid])

    return body(src_p, dst_p, asrc, adst)


def _sc_rows(src_p, dst_p, w_all, xp2, zrows, n, ept):
    nsup = ept // SUP
    rpt = ((n // NS + 7) // 8) * 8
    npad = NS * rpt
    mesh = plsc.VectorSubcoreMesh(core_axis_name="c", subcore_axis_name="s")

    @functools.partial(
        pl.kernel,
        mesh=mesh,
        compiler_params=pltpu.CompilerParams(needs_layout_passes=False),
        out_type=jax.ShapeDtypeStruct((2, npad, 128), jnp.float32),
        scratch_types=[
            pltpu.VMEM((NCH, CHUNK), jnp.int32),
            pltpu.VMEM((NCH, CHUNK), jnp.int32),
            pltpu.VMEM((NCH, CHUNK), jnp.float32),
            pltpu.VMEM((CHUNK,), jnp.int32),
            pltpu.VMEM((CHUNK,), jnp.int32),
            pltpu.VMEM((CHUNK,), jnp.int32),
            pltpu.VMEM((CHUNK,), jnp.int32),
            pltpu.VMEM((CHUNK, 128), jnp.float32),
            pltpu.VMEM((CHUNK, 128), jnp.float32),
            pltpu.VMEM_SHARED((npad, 128), jnp.float32),
            pltpu.SemaphoreType.DMA,
            pltpu.SemaphoreType.DMA,
        ],
    )
    def body(src_h, dst_h, w_all_h, xp2_h, zr_h, msg_h,
             src_t, dst_t, w_sup, gidx0, gidx1, didx0, didx1,
             rows0, rows1, acc_s, sem_g, sem_s):
        cid = lax.axis_index("c")
        sid = lax.axis_index("s")
        pltpu.sync_copy(zr_h, acc_s.at[pl.ds(sid * rpt, rpt)])
        plsc.subcore_barrier()

        cvec = jnp.full((LANES,), cid, dtype=jnp.int32)
        gbufs = [gidx0, gidx1]
        dbufs = [didx0, didx1]
        rbufs = [rows0, rows1]

        def build_idx(j):
            gb, db = gbufs[j % 2], dbufs[j % 2]
            for k in range(CHUNK // LANES):
                sv = src_t[j, pl.ds(k * LANES, LANES)]
                dv = dst_t[j, pl.ds(k * LANES, LANES)]
                gb[pl.ds(k * LANES, LANES)] = sv * 2 + cvec
                db[pl.ds(k * LANES, LANES)] = dv

        def fire_gather(j):
            pltpu.async_copy(xp2_h.at[gbufs[j % 2]], rbufs[j % 2], sem_g)

        def wait_gather(j):
            pltpu.make_async_copy(
                xp2_h.at[gbufs[j % 2]], rbufs[j % 2], sem_g).wait()

        def fire_scatter(j):
            pltpu.async_copy(rbufs[j % 2], acc_s.at[dbufs[j % 2]],
                             sem_s, add=True)

        def wait_scatter(j):
            pltpu.make_async_copy(rbufs[j % 2], acc_s.at[dbufs[j % 2]],
                                  sem_s).wait()

        def scale_rows(j):
            rb = rbufs[j % 2]

            @plsc.parallel_loop(0, CHUNK, step=1, unroll=4)
            def scale_row(r):
                ri = jnp.full((LANES,), r, dtype=jnp.int32)
                cj = jnp.full((LANES,), j, dtype=jnp.int32)
                wv = plsc.load_gather(w_sup, [cj, ri])
                for jj in range(128 // LANES):
                    sl = pl.ds(jj * LANES, LANES)
                    rb[r, sl] = rb[r, sl] * wv

        def sup_body(s, _):
            row0 = (sid * nsup + s) * NCH
            pltpu.sync_copy(src_h.at[pl.ds(row0, NCH)], src_t)
            pltpu.sync_copy(dst_h.at[pl.ds(row0, NCH)], dst_t)
            pltpu.sync_copy(w_all_h.at[cid, pl.ds(row0, NCH)], w_sup)
            for j in range(NCH):
                if j == 0:
                    build_idx(0)
                    fire_gather(0)
                if j > 0:
                    wait_scatter(j - 1)
                if j < NCH - 1:
                    build_idx(j + 1)
                    fire_gather(j + 1)
                wait_gather(j)
                scale_rows(j)
                fire_scatter(j)
            wait_scatter(NCH - 1)
            return 0

        lax.fori_loop(0, nsup, sup_body, 0)
        plsc.subcore_barrier()
        pltpu.sync_copy(acc_s.at[pl.ds(sid * rpt, rpt)],
                        msg_h.at[cid, pl.ds(sid * rpt, rpt)])

    return body(src_p, dst_p, w_all, xp2, zrows)


def _tc2_body(msg_ref, denp_ref, bc_ref, wa_ref, ba_ref, w1_ref, b1_ref,
              w2_ref, b2_ref, w3_ref, b3_ref, h3_ref):
    heads = []
    for h in range(2):
        den = jnp.sum(denp_ref[h], axis=0) + jnp.float32(1e-16)
        heads.append(msg_ref[h] / den[:, None])
    g = jnp.concatenate(heads, axis=1) + bc_ref[...]

    def lrelu(t):
        return jnp.where(t > 0, t, t * jnp.float32(0.01))

    g = lrelu(g)
    g = lrelu(jnp.dot(g, wa_ref[...], preferred_element_type=jnp.float32)
              + ba_ref[...])
    g = lrelu(jnp.dot(g, w1_ref[...], preferred_element_type=jnp.float32)
              + b1_ref[...])
    g = lrelu(jnp.dot(g, w2_ref[...], preferred_element_type=jnp.float32)
              + b2_ref[...])
    h3_ref[...] = jnp.dot(g, w3_ref[...], preferred_element_type=jnp.float32) \
        + b3_ref[...]


def _tc2(msg, denp, b_conv, Wa, ba, W1, b1, W2, b2, W3, b3):
    n = denp.shape[2]
    full = lambda shape: pl.BlockSpec(shape, lambda i: tuple(0 for _ in shape))
    return pl.pallas_call(
        _tc2_body,
        grid=(pl.cdiv(n, BLK),),
        in_specs=[
            pl.BlockSpec((2, BLK, 128), lambda i: (0, i, 0)),
            pl.BlockSpec((2, NS, BLK), lambda i: (0, 0, i)),
            full((1, 256)),
            full((256, 128)), full((1, 128)),
            full((128, 64)), full((1, 64)),
            full((64, 32)), full((1, 32)),
            full((32, 3)), full((1, 3)),
        ],
        out_specs=pl.BlockSpec((BLK, 3), lambda i: (i, 0)),
        out_shape=jax.ShapeDtypeStruct((n, 3), jnp.float32),
    )(msg, denp, b_conv.reshape(1, 256), Wa, ba.reshape(1, -1),
      W1, b1.reshape(1, -1), W2, b2.reshape(1, -1),
      W3, b3.reshape(1, -1))


def _tc3_body(hi_ref, hj_ref, out_ref):
    hi = hi_ref[...]
    hj = hj_ref[...]
    x2i = jnp.sum(hi * hi, axis=1)
    x2j = jnp.sum(hj * hj, axis=1)
    ip = lax.dot_general(hi, hj, (((1,), (1,)), ((), ())),
                         preferred_element_type=jnp.float32)
    d2 = x2i[:, None] + x2j[None, :] - 2.0 * ip
    d2 = jnp.maximum(d2, 0.0)
    pos = d2 > 0
    out_ref[...] = jnp.where(pos, jnp.sqrt(jnp.where(pos, d2, 1.0)), 0.0)


def _tc3(h3):
    n = h3.shape[0]
    br, bc = 256, 2048
    return pl.pallas_call(
        _tc3_body,
        grid=(pl.cdiv(n, br), pl.cdiv(n, bc)),
        in_specs=[
            pl.BlockSpec((br, 3), lambda i, j: (i, 0)),
            pl.BlockSpec((bc, 3), lambda i, j: (j, 0)),
        ],
        out_specs=pl.BlockSpec((br, bc), lambda i, j: (i, j)),
        out_shape=jax.ShapeDtypeStruct((n, n), jnp.float32),
    )(h3, h3)


def kernel(x, edge_index, W, att_src, att_dst, b_conv,
           Wa, ba, W1, b1, W2, b2, W3, b3):
    n = x.shape[0]
    e = edge_index.shape[1]

    xp, asrc, adst = _tc1(x, W, att_src.reshape(2, 128),
                          att_dst.reshape(2, 128))
    xp2 = xp.reshape(2 * n, 128)

    ept = ((e + NS * SUP - 1) // (NS * SUP)) * SUP
    e_pad = NS * ept
    src = edge_index[0].astype(jnp.int32)
    dst = edge_index[1].astype(jnp.int32)
    pad = jnp.zeros((e_pad - e,), dtype=jnp.int32)
    src_p = jnp.concatenate([src, pad]).reshape(-1, CHUNK)
    dst_p = jnp.concatenate([dst, pad]).reshape(-1, CHUNK)
    rpt = ((n // NS + 7) // 8) * 8
    zrows = jnp.zeros((rpt, 128), dtype=jnp.float32)

    w_all, denp = _sc_weights(src_p, dst_p, asrc, adst, n, e, ept)
    msg = _sc_rows(src_p, dst_p, w_all, xp2, zrows, n, ept)

    h3 = _tc2(msg, denp, b_conv, Wa, ba, W1, b1, W2, b2, W3, b3)
    return _tc3(h3)


# speedup vs baseline: 1.7986x; 1.1297x over previous
import functools

import jax
import jax.numpy as jnp
from jax import lax
from jax.experimental import pallas as pl
from jax.experimental.pallas import tpu as pltpu
from jax.experimental.pallas import tpu_sc as plsc

NS = 16
LANES = 16
CHUNK = 128
NCH = 16
SUP = NCH * CHUNK
BLK = 2048


def _tc1_body(x_ref, w_ref, as_ref, ad_ref, xp_ref, asrc_ref, adst_ref):
    xb = jnp.dot(x_ref[...], w_ref[...], preferred_element_type=jnp.float32)
    xp_ref[...] = xb
    ss, dd = [], []
    for h in range(2):
        blk = xb[:, h * 128:(h + 1) * 128]
        ss.append(jnp.sum(blk * as_ref[h, :][None, :], axis=1)[None, :])
        dd.append(jnp.sum(blk * ad_ref[h, :][None, :], axis=1)[None, :])
    asrc_ref[...] = jnp.concatenate(ss, axis=0)
    adst_ref[...] = jnp.concatenate(dd, axis=0)


def _tc1(x, W, att_s, att_d):
    n = x.shape[0]
    f = x.shape[1]
    return pl.pallas_call(
        _tc1_body,
        grid=(pl.cdiv(n, BLK),),
        in_specs=[
            pl.BlockSpec((BLK, f), lambda i: (i, 0)),
            pl.BlockSpec((f, 256), lambda i: (0, 0)),
            pl.BlockSpec((2, 128), lambda i: (0, 0)),
            pl.BlockSpec((2, 128), lambda i: (0, 0)),
        ],
        out_specs=[
            pl.BlockSpec((BLK, 256), lambda i: (i, 0)),
            pl.BlockSpec((2, BLK), lambda i: (0, i)),
            pl.BlockSpec((2, BLK), lambda i: (0, i)),
        ],
        out_shape=[
            jax.ShapeDtypeStruct((n, 256), jnp.float32),
            jax.ShapeDtypeStruct((2, n), jnp.float32),
            jax.ShapeDtypeStruct((2, n), jnp.float32),
        ],
    )(x, W, att_s, att_d)


def _sc_weights(src_p, dst_p, asrc, adst, n, e_real, ept):
    nsup = ept // SUP
    mesh = plsc.VectorSubcoreMesh(core_axis_name="c", subcore_axis_name="s")

    @functools.partial(
        pl.kernel,
        mesh=mesh,
        compiler_params=pltpu.CompilerParams(needs_layout_passes=False),
        out_type=(
            jax.ShapeDtypeStruct((2, NS * ept // CHUNK, CHUNK), jnp.float32),
            jax.ShapeDtypeStruct((2, NS, n), jnp.float32),
        ),
        scratch_types=[
            pltpu.VMEM((n,), jnp.float32),
            pltpu.VMEM((n,), jnp.float32),
            pltpu.VMEM((n,), jnp.float32),
            pltpu.VMEM((NCH, CHUNK), jnp.int32),
            pltpu.VMEM((NCH, CHUNK), jnp.int32),
            pltpu.VMEM((NCH, CHUNK), jnp.float32),
            pltpu.VMEM((LANES,), jnp.int32),
            pltpu.VMEM((LANES,), jnp.float32),
        ],
    )
    def body(src_h, dst_h, asrc_h, adst_h, w_all_h, denp_h,
             asrc_t, adst_t, den_t, src_t, dst_t, w_sup, kbuf, wbuf):
        cid = lax.axis_index("c")
        sid = lax.axis_index("s")
        e0 = sid * ept
        pltpu.sync_copy(asrc_h.at[cid], asrc_t)
        pltpu.sync_copy(adst_h.at[cid], adst_t)

        zv = jnp.zeros((LANES,), jnp.float32)

        def zb(i, _):
            den_t[pl.ds(i * LANES, LANES)] = zv
            return 0
        lax.fori_loop(0, n // LANES, zb, 0)

        iota = lax.iota(jnp.int32, LANES)
        shifts = [1, 2, 4, 8]
        shift_idx = [jnp.maximum(iota - s, 0) for s in shifts]
        next_idx = jnp.minimum(iota + 1, LANES - 1)

        def sup_body(s, _):
            row0 = (sid * nsup + s) * NCH
            pltpu.sync_copy(src_h.at[pl.ds(row0, NCH)], src_t)
            pltpu.sync_copy(dst_h.at[pl.ds(row0, NCH)], dst_t)
            for j in range(NCH):
                for k in range(CHUNK // LANES):
                    sv = src_t[j, pl.ds(k * LANES, LANES)]
                    dv = dst_t[j, pl.ds(k * LANES, LANES)]
                    al = plsc.load_gather(asrc_t, [sv]) + \
                        plsc.load_gather(adst_t, [dv])
                    al = jnp.where(al > 0, al, al * jnp.float32(0.2))
                    w = jnp.exp(al)
                    egid = e0 + s * SUP + j * CHUNK + k * LANES + iota
                    w = jnp.where(egid < e_real, w, jnp.float32(0.0))
                    w_sup[j, pl.ds(k * LANES, LANES)] = w
                    ks_, vs_ = plsc.sort_key_val(dv, w)
                    kbuf[...] = ks_
                    for si, sh in enumerate(shifts):
                        wbuf[...] = vs_
                        kprev = plsc.load_gather(kbuf, [shift_idx[si]])
                        vprev = plsc.load_gather(wbuf, [shift_idx[si]])
                        ok = (iota >= sh) & (kprev == ks_)
                        vs_ = vs_ + jnp.where(ok, vprev, jnp.float32(0.0))
                    knext = plsc.load_gather(kbuf, [next_idx])
                    ends = (iota == LANES - 1) | (knext != ks_)
                    plsc.addupdate_scatter(den_t, [ks_], vs_, mask=ends)
            pltpu.sync_copy(w_sup, w_all_h.at[cid, pl.ds(row0, NCH)])
            return 0

        lax.fori_loop(0, nsup, sup_body, 0)
        pltpu.sync_copy(den_t, denp_h.at[cid, sid])

    return body(src_p, dst_p, asrc, adst)


def _sc_rows(src_p, dst_p, w_all, xp2, zrows, n, ept):
    nsup = ept // SUP
    rpt = ((n // NS + 7) // 8) * 8
    npad = NS * rpt
    mesh = plsc.VectorSubcoreMesh(core_axis_name="c", subcore_axis_name="s")

    @functools.partial(
        pl.kernel,
        mesh=mesh,
        compiler_params=pltpu.CompilerParams(needs_layout_passes=False),
        out_type=jax.ShapeDtypeStruct((2, npad, 128), jnp.float32),
        scratch_types=[
            pltpu.VMEM((NCH, CHUNK), jnp.int32),
            pltpu.VMEM((NCH, CHUNK), jnp.int32),
            pltpu.VMEM((NCH, CHUNK), jnp.float32),
            pltpu.VMEM((CHUNK,), jnp.int32),
            pltpu.VMEM((CHUNK,), jnp.int32),
            pltpu.VMEM((CHUNK,), jnp.int32),
            pltpu.VMEM((CHUNK,), jnp.int32),
            pltpu.VMEM((CHUNK, 128), jnp.float32),
            pltpu.VMEM((CHUNK, 128), jnp.float32),
            pltpu.VMEM_SHARED((npad, 128), jnp.float32),
            pltpu.SemaphoreType.DMA,
            pltpu.SemaphoreType.DMA,
        ],
    )
    def body(src_h, dst_h, w_all_h, xp2_h, zr_h, msg_h,
             src_t, dst_t, w_sup, gidx0, gidx1, didx0, didx1,
             rows0, rows1, acc_s, sem_g, sem_s):
        cid = lax.axis_index("c")
        sid = lax.axis_index("s")
        pltpu.sync_copy(zr_h, acc_s.at[pl.ds(sid * rpt, rpt)])
        plsc.subcore_barrier()

        cvec = jnp.full((LANES,), cid, dtype=jnp.int32)
        gbufs = [gidx0, gidx1]
        dbufs = [didx0, didx1]
        rbufs = [rows0, rows1]

        def build_idx(j):
            gb, db = gbufs[j % 2], dbufs[j % 2]
            for k in range(CHUNK // LANES):
                sv = src_t[j, pl.ds(k * LANES, LANES)]
                dv = dst_t[j, pl.ds(k * LANES, LANES)]
                gb[pl.ds(k * LANES, LANES)] = sv * 2 + cvec
                db[pl.ds(k * LANES, LANES)] = dv

        def fire_gather(j):
            pltpu.async_copy(xp2_h.at[gbufs[j % 2]], rbufs[j % 2], sem_g)

        def wait_gather(j):
            pltpu.make_async_copy(
                xp2_h.at[gbufs[j % 2]], rbufs[j % 2], sem_g).wait()

        def fire_scatter(j):
            pltpu.async_copy(rbufs[j % 2], acc_s.at[dbufs[j % 2]],
                             sem_s, add=True)

        def wait_scatter(j):
            pltpu.make_async_copy(rbufs[j % 2], acc_s.at[dbufs[j % 2]],
                                  sem_s).wait()

        def scale_rows(j):
            rb = rbufs[j % 2]

            @plsc.parallel_loop(0, CHUNK, step=1, unroll=4)
            def scale_row(r):
                ri = jnp.full((LANES,), r, dtype=jnp.int32)
                cj = jnp.full((LANES,), j, dtype=jnp.int32)
                wv = plsc.load_gather(w_sup, [cj, ri])
                for jj in range(128 // LANES):
                    sl = pl.ds(jj * LANES, LANES)
                    rb[r, sl] = rb[r, sl] * wv

        def sup_body(s, _):
            row0 = (sid * nsup + s) * NCH
            pltpu.sync_copy(src_h.at[pl.ds(row0, NCH)], src_t)
            pltpu.sync_copy(dst_h.at[pl.ds(row0, NCH)], dst_t)
            pltpu.sync_copy(w_all_h.at[cid, pl.ds(row0, NCH)], w_sup)
            for j in range(NCH):
                if j == 0:
                    build_idx(0)
                    fire_gather(0)
                if j > 0:
                    wait_scatter(j - 1)
                if j < NCH - 1:
                    build_idx(j + 1)
                    fire_gather(j + 1)
                wait_gather(j)
                scale_rows(j)
                fire_scatter(j)
            wait_scatter(NCH - 1)
            return 0

        lax.fori_loop(0, nsup, sup_body, 0)
        plsc.subcore_barrier()
        pltpu.sync_copy(acc_s.at[pl.ds(sid * rpt, rpt)],
                        msg_h.at[cid, pl.ds(sid * rpt, rpt)])

    return body(src_p, dst_p, w_all, xp2, zrows)


def _tc2_body(msg_ref, denp_ref, bc_ref, wa_ref, ba_ref, w1_ref, b1_ref,
              w2_ref, b2_ref, w3_ref, b3_ref, h3_ref):
    heads = []
    for h in range(2):
        den = jnp.sum(denp_ref[h], axis=0) + jnp.float32(1e-16)
        heads.append(msg_ref[h] / den[:, None])
    g = jnp.concatenate(heads, axis=1) + bc_ref[...]

    def lrelu(t):
        return jnp.where(t > 0, t, t * jnp.float32(0.01))

    g = lrelu(g)
    g = lrelu(jnp.dot(g, wa_ref[...], preferred_element_type=jnp.float32)
              + ba_ref[...])
    g = lrelu(jnp.dot(g, w1_ref[...], preferred_element_type=jnp.float32)
              + b1_ref[...])
    g = lrelu(jnp.dot(g, w2_ref[...], preferred_element_type=jnp.float32)
              + b2_ref[...])
    h3_ref[...] = jnp.dot(g, w3_ref[...], preferred_element_type=jnp.float32) \
        + b3_ref[...]


def _tc2(msg, denp, b_conv, Wa, ba, W1, b1, W2, b2, W3, b3):
    n = denp.shape[2]
    full = lambda shape: pl.BlockSpec(shape, lambda i: tuple(0 for _ in shape))
    return pl.pallas_call(
        _tc2_body,
        grid=(pl.cdiv(n, BLK),),
        in_specs=[
            pl.BlockSpec((2, BLK, 128), lambda i: (0, i, 0)),
            pl.BlockSpec((2, NS, BLK), lambda i: (0, 0, i)),
            full((1, 256)),
            full((256, 128)), full((1, 128)),
            full((128, 64)), full((1, 64)),
            full((64, 32)), full((1, 32)),
            full((32, 3)), full((1, 3)),
        ],
        out_specs=pl.BlockSpec((BLK, 3), lambda i: (i, 0)),
        out_shape=jax.ShapeDtypeStruct((n, 3), jnp.float32),
    )(msg, denp, b_conv.reshape(1, 256), Wa, ba.reshape(1, -1),
      W1, b1.reshape(1, -1), W2, b2.reshape(1, -1),
      W3, b3.reshape(1, -1))


def _tc3_body(hi_ref, hj_ref, out_ref):
    hi = hi_ref[...]
    hj = hj_ref[...]
    x2i = jnp.sum(hi * hi, axis=1)
    x2j = jnp.sum(hj * hj, axis=1)
    ip = lax.dot_general(hi, hj, (((1,), (1,)), ((), ())),
                         preferred_element_type=jnp.float32)
    d2 = x2i[:, None] + x2j[None, :] - 2.0 * ip
    d2 = jnp.maximum(d2, 0.0)
    pos = d2 > 0
    out_ref[...] = jnp.where(pos, jnp.sqrt(jnp.where(pos, d2, 1.0)), 0.0)


def _tc3(h3):
    n = h3.shape[0]
    br, bc = 512, 2048
    return pl.pallas_call(
        _tc3_body,
        grid=(pl.cdiv(n, br), pl.cdiv(n, bc)),
        in_specs=[
            pl.BlockSpec((br, 3), lambda i, j: (i, 0)),
            pl.BlockSpec((bc, 3), lambda i, j: (j, 0)),
        ],
        out_specs=pl.BlockSpec((br, bc), lambda i, j: (i, j)),
        out_shape=jax.ShapeDtypeStruct((n, n), jnp.float32),
    )(h3, h3)


def kernel(x, edge_index, W, att_src, att_dst, b_conv,
           Wa, ba, W1, b1, W2, b2, W3, b3):
    n = x.shape[0]
    e = edge_index.shape[1]

    xp, asrc, adst = _tc1(x, W, att_src.reshape(2, 128),
                          att_dst.reshape(2, 128))
    xp2 = xp.reshape(2 * n, 128)

    ept = ((e + NS * SUP - 1) // (NS * SUP)) * SUP
    e_pad = NS * ept
    src = edge_index[0].astype(jnp.int32)
    dst = edge_index[1].astype(jnp.int32)
    pad = jnp.zeros((e_pad - e,), dtype=jnp.int32)
    src_p = jnp.concatenate([src, pad]).reshape(-1, CHUNK)
    dst_p = jnp.concatenate([dst, pad]).reshape(-1, CHUNK)
    rpt = ((n // NS + 7) // 8) * 8
    zrows = jnp.zeros((rpt, 128), dtype=jnp.float32)

    w_all, denp = _sc_weights(src_p, dst_p, asrc, adst, n, e, ept)
    msg = _sc_rows(src_p, dst_p, w_all, xp2, zrows, n, ept)

    h3 = _tc2(msg, denp, b_conv, Wa, ba, W1, b1, W2, b2, W3, b3)
    return _tc3(h3)


# speedup vs baseline: 1.8463x; 1.0265x over previous
import functools

import jax
import jax.numpy as jnp
from jax import lax
from jax.experimental import pallas as pl
from jax.experimental.pallas import tpu as pltpu
from jax.experimental.pallas import tpu_sc as plsc

NS = 16
LANES = 16
CHUNK = 128
NCH = 16
SUP = NCH * CHUNK
BLK = 2048


def _tc1_body(x_ref, w_ref, as_ref, ad_ref, xp_ref, asrc_ref, adst_ref):
    xb = jnp.dot(x_ref[...], w_ref[...], preferred_element_type=jnp.float32)
    xp_ref[...] = xb
    ss, dd = [], []
    for h in range(2):
        blk = xb[:, h * 128:(h + 1) * 128]
        ss.append(jnp.sum(blk * as_ref[h, :][None, :], axis=1)[None, :])
        dd.append(jnp.sum(blk * ad_ref[h, :][None, :], axis=1)[None, :])
    asrc_ref[...] = jnp.concatenate(ss, axis=0)
    adst_ref[...] = jnp.concatenate(dd, axis=0)


def _tc1(x, W, att_s, att_d):
    n = x.shape[0]
    f = x.shape[1]
    return pl.pallas_call(
        _tc1_body,
        grid=(pl.cdiv(n, BLK),),
        in_specs=[
            pl.BlockSpec((BLK, f), lambda i: (i, 0)),
            pl.BlockSpec((f, 256), lambda i: (0, 0)),
            pl.BlockSpec((2, 128), lambda i: (0, 0)),
            pl.BlockSpec((2, 128), lambda i: (0, 0)),
        ],
        out_specs=[
            pl.BlockSpec((BLK, 256), lambda i: (i, 0)),
            pl.BlockSpec((2, BLK), lambda i: (0, i)),
            pl.BlockSpec((2, BLK), lambda i: (0, i)),
        ],
        out_shape=[
            jax.ShapeDtypeStruct((n, 256), jnp.float32),
            jax.ShapeDtypeStruct((2, n), jnp.float32),
            jax.ShapeDtypeStruct((2, n), jnp.float32),
        ],
    )(x, W, att_s, att_d)


def _sc_weights(src_p, dst_p, asrc, adst, n, e_real, ept):
    nsup = ept // SUP
    mesh = plsc.VectorSubcoreMesh(core_axis_name="c", subcore_axis_name="s")

    @functools.partial(
        pl.kernel,
        mesh=mesh,
        compiler_params=pltpu.CompilerParams(needs_layout_passes=False),
        out_type=(
            jax.ShapeDtypeStruct((2, NS * ept // CHUNK, CHUNK), jnp.float32),
            jax.ShapeDtypeStruct((2, NS, n), jnp.float32),
        ),
        scratch_types=[
            pltpu.VMEM((n,), jnp.float32),
            pltpu.VMEM((n,), jnp.float32),
            pltpu.VMEM((n,), jnp.float32),
            pltpu.VMEM((NCH, CHUNK), jnp.int32),
            pltpu.VMEM((NCH, CHUNK), jnp.int32),
            pltpu.VMEM((NCH, CHUNK), jnp.float32),
            pltpu.VMEM((LANES,), jnp.int32),
            pltpu.VMEM((LANES,), jnp.float32),
        ],
    )
    def body(src_h, dst_h, asrc_h, adst_h, w_all_h, denp_h,
             asrc_t, adst_t, den_t, src_t, dst_t, w_sup, kbuf, wbuf):
        cid = lax.axis_index("c")
        sid = lax.axis_index("s")
        e0 = sid * ept
        pltpu.sync_copy(asrc_h.at[cid], asrc_t)
        pltpu.sync_copy(adst_h.at[cid], adst_t)

        zv = jnp.zeros((LANES,), jnp.float32)

        def zb(i, _):
            den_t[pl.ds(i * LANES, LANES)] = zv
            return 0
        lax.fori_loop(0, n // LANES, zb, 0)

        iota = lax.iota(jnp.int32, LANES)
        shifts = [1, 2, 4, 8]
        shift_idx = [jnp.maximum(iota - s, 0) for s in shifts]
        next_idx = jnp.minimum(iota + 1, LANES - 1)

        def sup_body(s, _):
            row0 = (sid * nsup + s) * NCH
            pltpu.sync_copy(src_h.at[pl.ds(row0, NCH)], src_t)
            pltpu.sync_copy(dst_h.at[pl.ds(row0, NCH)], dst_t)
            for j in range(NCH):
                for k in range(CHUNK // LANES):
                    sv = src_t[j, pl.ds(k * LANES, LANES)]
                    dv = dst_t[j, pl.ds(k * LANES, LANES)]
                    al = plsc.load_gather(asrc_t, [sv]) + \
                        plsc.load_gather(adst_t, [dv])
                    al = jnp.where(al > 0, al, al * jnp.float32(0.2))
                    w = jnp.exp(al)
                    egid = e0 + s * SUP + j * CHUNK + k * LANES + iota
                    w = jnp.where(egid < e_real, w, jnp.float32(0.0))
                    w_sup[j, pl.ds(k * LANES, LANES)] = w
                    ks_, vs_ = plsc.sort_key_val(dv, w)
                    kbuf[...] = ks_
                    for si, sh in enumerate(shifts):
                        wbuf[...] = vs_
                        kprev = plsc.load_gather(kbuf, [shift_idx[si]])
                        vprev = plsc.load_gather(wbuf, [shift_idx[si]])
                        ok = (iota >= sh) & (kprev == ks_)
                        vs_ = vs_ + jnp.where(ok, vprev, jnp.float32(0.0))
                    knext = plsc.load_gather(kbuf, [next_idx])
                    ends = (iota == LANES - 1) | (knext != ks_)
                    plsc.addupdate_scatter(den_t, [ks_], vs_, mask=ends)
            pltpu.sync_copy(w_sup, w_all_h.at[cid, pl.ds(row0, NCH)])
            return 0

        lax.fori_loop(0, nsup, sup_body, 0)
        pltpu.sync_copy(den_t, denp_h.at[cid, sid])

    return body(src_p, dst_p, asrc, adst)


def _sc_rows(src_p, dst_p, w_all, xp2, zrows, n, ept):
    nsup = ept // SUP
    rpt = ((n // NS + 7) // 8) * 8
    npad = NS * rpt
    mesh = plsc.VectorSubcoreMesh(core_axis_name="c", subcore_axis_name="s")

    @functools.partial(
        pl.kernel,
        mesh=mesh,
        compiler_params=pltpu.CompilerParams(needs_layout_passes=False),
        out_type=jax.ShapeDtypeStruct((2, npad, 128), jnp.float32),
        scratch_types=[
            pltpu.VMEM((NCH, CHUNK), jnp.int32),
            pltpu.VMEM((NCH, CHUNK), jnp.int32),
            pltpu.VMEM((NCH, CHUNK), jnp.float32),
            pltpu.VMEM((CHUNK,), jnp.int32),
            pltpu.VMEM((CHUNK,), jnp.int32),
            pltpu.VMEM((CHUNK,), jnp.int32),
            pltpu.VMEM((CHUNK,), jnp.int32),
            pltpu.VMEM((CHUNK, 128), jnp.float32),
            pltpu.VMEM((CHUNK, 128), jnp.float32),
            pltpu.VMEM_SHARED((npad, 128), jnp.float32),
            pltpu.SemaphoreType.DMA,
            pltpu.SemaphoreType.DMA,
        ],
    )
    def body(src_h, dst_h, w_all_h, xp2_h, zr_h, msg_h,
             src_t, dst_t, w_sup, gidx0, gidx1, didx0, didx1,
             rows0, rows1, acc_s, sem_g, sem_s):
        cid = lax.axis_index("c")
        sid = lax.axis_index("s")
        pltpu.sync_copy(zr_h, acc_s.at[pl.ds(sid * rpt, rpt)])
        plsc.subcore_barrier()

        cvec = jnp.full((LANES,), cid, dtype=jnp.int32)
        gbufs = [gidx0, gidx1]
        dbufs = [didx0, didx1]
        rbufs = [rows0, rows1]

        def build_idx(j):
            gb, db = gbufs[j % 2], dbufs[j % 2]
            for k in range(CHUNK // LANES):
                sv = src_t[j, pl.ds(k * LANES, LANES)]
                dv = dst_t[j, pl.ds(k * LANES, LANES)]
                gb[pl.ds(k * LANES, LANES)] = sv * 2 + cvec
                db[pl.ds(k * LANES, LANES)] = dv

        def fire_gather(j):
            pltpu.async_copy(xp2_h.at[gbufs[j % 2]], rbufs[j % 2], sem_g)

        def wait_gather(j):
            pltpu.make_async_copy(
                xp2_h.at[gbufs[j % 2]], rbufs[j % 2], sem_g).wait()

        def fire_scatter(j):
            pltpu.async_copy(rbufs[j % 2], acc_s.at[dbufs[j % 2]],
                             sem_s, add=True)

        def wait_scatter(j):
            pltpu.make_async_copy(rbufs[j % 2], acc_s.at[dbufs[j % 2]],
                                  sem_s).wait()

        def scale_rows(j):
            rb = rbufs[j % 2]

            @plsc.parallel_loop(0, CHUNK, step=1, unroll=4)
            def scale_row(r):
                ri = jnp.full((LANES,), r, dtype=jnp.int32)
                cj = jnp.full((LANES,), j, dtype=jnp.int32)
                wv = plsc.load_gather(w_sup, [cj, ri])
                for jj in range(128 // LANES):
                    sl = pl.ds(jj * LANES, LANES)
                    rb[r, sl] = rb[r, sl] * wv

        def sup_body(s, _):
            row0 = (sid * nsup + s) * NCH
            pltpu.sync_copy(src_h.at[pl.ds(row0, NCH)], src_t)
            pltpu.sync_copy(dst_h.at[pl.ds(row0, NCH)], dst_t)
            pltpu.sync_copy(w_all_h.at[cid, pl.ds(row0, NCH)], w_sup)
            for j in range(NCH):
                if j == 0:
                    build_idx(0)
                    fire_gather(0)
                if j > 0:
                    wait_scatter(j - 1)
                if j < NCH - 1:
                    build_idx(j + 1)
                    fire_gather(j + 1)
                wait_gather(j)
                scale_rows(j)
                fire_scatter(j)
            wait_scatter(NCH - 1)
            return 0

        lax.fori_loop(0, nsup, sup_body, 0)
        plsc.subcore_barrier()
        pltpu.sync_copy(acc_s.at[pl.ds(sid * rpt, rpt)],
                        msg_h.at[cid, pl.ds(sid * rpt, rpt)])

    return body(src_p, dst_p, w_all, xp2, zrows)


def _tc2_body(msg_ref, denp_ref, bc_ref, wa_ref, ba_ref, w1_ref, b1_ref,
              w2_ref, b2_ref, w3_ref, b3_ref, h3_ref):
    heads = []
    for h in range(2):
        den = jnp.sum(denp_ref[h], axis=0) + jnp.float32(1e-16)
        heads.append(msg_ref[h] / den[:, None])
    g = jnp.concatenate(heads, axis=1) + bc_ref[...]

    def lrelu(t):
        return jnp.where(t > 0, t, t * jnp.float32(0.01))

    g = lrelu(g)
    g = lrelu(jnp.dot(g, wa_ref[...], preferred_element_type=jnp.float32)
              + ba_ref[...])
    g = lrelu(jnp.dot(g, w1_ref[...], preferred_element_type=jnp.float32)
              + b1_ref[...])
    g = lrelu(jnp.dot(g, w2_ref[...], preferred_element_type=jnp.float32)
              + b2_ref[...])
    h3_ref[...] = jnp.dot(g, w3_ref[...], preferred_element_type=jnp.float32) \
        + b3_ref[...]


def _tc2(msg, denp, b_conv, Wa, ba, W1, b1, W2, b2, W3, b3):
    n = denp.shape[2]
    full = lambda shape: pl.BlockSpec(shape, lambda i: tuple(0 for _ in shape))
    return pl.pallas_call(
        _tc2_body,
        grid=(pl.cdiv(n, BLK),),
        in_specs=[
            pl.BlockSpec((2, BLK, 128), lambda i: (0, i, 0)),
            pl.BlockSpec((2, NS, BLK), lambda i: (0, 0, i)),
            full((1, 256)),
            full((256, 128)), full((1, 128)),
            full((128, 64)), full((1, 64)),
            full((64, 32)), full((1, 32)),
            full((32, 3)), full((1, 3)),
        ],
        out_specs=pl.BlockSpec((BLK, 3), lambda i: (i, 0)),
        out_shape=jax.ShapeDtypeStruct((n, 3), jnp.float32),
    )(msg, denp, b_conv.reshape(1, 256), Wa, ba.reshape(1, -1),
      W1, b1.reshape(1, -1), W2, b2.reshape(1, -1),
      W3, b3.reshape(1, -1))


def _tc3_body(hi_ref, hj_ref, out_ref):
    hi = hi_ref[...]
    hj = hj_ref[...]
    x2i = jnp.sum(hi * hi, axis=1)
    x2j = jnp.sum(hj * hj, axis=1)
    ip = lax.dot_general(hi, hj, (((1,), (1,)), ((), ())),
                         preferred_element_type=jnp.float32)
    d2 = x2i[:, None] + x2j[None, :] - 2.0 * ip
    d2 = jnp.maximum(d2, 0.0)
    pos = d2 > 0
    out_ref[...] = jnp.where(pos, jnp.sqrt(jnp.where(pos, d2, 1.0)), 0.0)


def _tc3(h3):
    n = h3.shape[0]
    br, bc = 1024, 2048
    return pl.pallas_call(
        _tc3_body,
        grid=(pl.cdiv(n, br), pl.cdiv(n, bc)),
        in_specs=[
            pl.BlockSpec((br, 3), lambda i, j: (i, 0)),
            pl.BlockSpec((bc, 3), lambda i, j: (j, 0)),
        ],
        out_specs=pl.BlockSpec((br, bc), lambda i, j: (i, j)),
        out_shape=jax.ShapeDtypeStruct((n, n), jnp.float32),
    )(h3, h3)


def kernel(x, edge_index, W, att_src, att_dst, b_conv,
           Wa, ba, W1, b1, W2, b2, W3, b3):
    n = x.shape[0]
    e = edge_index.shape[1]

    xp, asrc, adst = _tc1(x, W, att_src.reshape(2, 128),
                          att_dst.reshape(2, 128))
    xp2 = xp.reshape(2 * n, 128)

    ept = ((e + NS * SUP - 1) // (NS * SUP)) * SUP
    e_pad = NS * ept
    src = edge_index[0].astype(jnp.int32)
    dst = edge_index[1].astype(jnp.int32)
    pad = jnp.zeros((e_pad - e,), dtype=jnp.int32)
    src_p = jnp.concatenate([src, pad]).reshape(-1, CHUNK)
    dst_p = jnp.concatenate([dst, pad]).reshape(-1, CHUNK)
    rpt = ((n // NS + 7) // 8) * 8
    zrows = jnp.zeros((rpt, 128), dtype=jnp.float32)

    w_all, denp = _sc_weights(src_p, dst_p, asrc, adst, n, e, ept)
    msg = _sc_rows(src_p, dst_p, w_all, xp2, zrows, n, ept)

    h3 = _tc2(msg, denp, b_conv, Wa, ba, W1, b1, W2, b2, W3, b3)
    return _tc3(h3)
